# baseline (speedup 1.0000x reference)
import jax
import jax.numpy as jnp
from jax.experimental import pallas as pl
from jax.experimental.pallas import tpu as pltpu

BS, SEQ, D, PLEN = 256, 196, 768, 10
OUT_SEQ = PLEN + SEQ


def _body(tid_ref, x_ref, pool_ref, out_ref):
    out_ref[0, :PLEN, :] = pool_ref[0]
    out_ref[0, PLEN:, :] = x_ref[0]


def kernel(x, prompt_pool, task_id):
    grid_spec = pltpu.PrefetchScalarGridSpec(
        num_scalar_prefetch=1,
        grid=(BS,),
        in_specs=[
            pl.BlockSpec((1, SEQ, D), lambda i, tid: (i, 0, 0)),
            pl.BlockSpec((1, PLEN, D), lambda i, tid: (tid[i], 0, 0)),
        ],
        out_specs=pl.BlockSpec((1, OUT_SEQ, D), lambda i, tid: (i, 0, 0)),
    )
    return pl.pallas_call(
        _body,
        grid_spec=grid_spec,
        out_shape=jax.ShapeDtypeStruct((BS, OUT_SEQ, D), x.dtype),
    )(task_id.astype(jnp.int32), x, prompt_pool)
